# baseline (speedup 1.0000x reference)
import functools

import jax
import jax.numpy as jnp
from jax import lax
from jax.experimental import pallas as pl
from jax.experimental.pallas import tpu as pltpu
from jax.experimental.pallas import tpu_sc as plsc

N = 10000
E = 160000
K = 4
H = 64
PE = 32

NPAD = 10240
EPAD = 163840
EB = 64
NBUF = 4
NCH = 2 * K
NPAIR = NCH // 2
W = 2 * H
ROWS_PER_TILE = NPAD // 16
DUMP = N


@functools.cache
def _mesh():
    return plsc.VectorSubcoreMesh(core_axis_name="c", subcore_axis_name="s")


def _pipelined_scatter(table_hbm, src_v, dst_v, rows_v, acc_sh, sem, nb):
    for i in range(NBUF):
        pltpu.async_copy(table_hbm.at[src_v.at[i]], rows_v.at[i], sem.at[i])

    @pl.loop(0, nb)
    def _(b):
        buf = lax.rem(b, NBUF)
        pltpu.make_async_copy(table_hbm.at[src_v.at[b]], rows_v.at[buf],
                              sem.at[buf]).wait()
        pltpu.sync_copy(rows_v.at[buf], acc_sh.at[dst_v.at[b]], add=True)

        @pl.when(b + NBUF < nb)
        def _():
            pltpu.async_copy(table_hbm.at[src_v.at[b + NBUF]], rows_v.at[buf],
                             sem.at[buf])


def _sc_agg0(xpad, src0, dst0, zeros):
    nb = src0.shape[2]
    seg = 16

    @functools.partial(
        pl.kernel,
        out_type=jax.ShapeDtypeStruct((2, NPAD, W), jnp.float32),
        mesh=_mesh(),
        scratch_types=[
            pltpu.VMEM((seg, EB), jnp.int32),
            pltpu.VMEM((seg, EB), jnp.int32),
            pltpu.VMEM((NBUF, EB, W), jnp.float32),
            pltpu.VMEM_SHARED((NPAD, W), jnp.float32),
            pltpu.SemaphoreType.DMA((NBUF,)),
        ],
    )
    def body(x_hbm, src_hbm, dst_hbm, z_hbm, out_hbm, src_v, dst_v, rows_v,
             acc_sh, sem):
        cid = lax.axis_index("c")
        sid = lax.axis_index("s")
        sl = pl.ds(sid * ROWS_PER_TILE, ROWS_PER_TILE)

        @pl.when(cid == 0)
        def _():
            pltpu.sync_copy(x_hbm.at[sl], acc_sh.at[sl])

        @pl.when(cid != 0)
        def _():
            pltpu.sync_copy(z_hbm, acc_sh.at[sl])

        plsc.subcore_barrier()
        for s in range(nb // seg):
            pltpu.sync_copy(src_hbm.at[cid, sid, pl.ds(s * seg, seg)], src_v)
            pltpu.sync_copy(dst_hbm.at[cid, sid, pl.ds(s * seg, seg)], dst_v)
            _pipelined_scatter(x_hbm, src_v, dst_v, rows_v, acc_sh, sem, seg)
        plsc.subcore_barrier()
        pltpu.sync_copy(acc_sh.at[sl], out_hbm.at[cid, sl])

    return body(xpad, src0, dst0, zeros)


def _sc_agg(h_flat, srcb, dst2):
    nb = dst2.shape[1]
    seg = 32

    @functools.partial(
        pl.kernel,
        out_type=jax.ShapeDtypeStruct((2, NPAD, W), jnp.float32),
        mesh=_mesh(),
        scratch_types=[
            pltpu.VMEM((seg, EB), jnp.int32),
            pltpu.VMEM((seg, EB), jnp.int32),
            pltpu.VMEM((NBUF, EB, W), jnp.float32),
            pltpu.VMEM_SHARED((NPAD, W), jnp.float32),
            pltpu.SemaphoreType.DMA((NBUF,)),
        ],
    )
    def body(h_hbm, src_hbm, dst_hbm, out_hbm, src_v, dst_v, rows_v,
             acc_sh, sem):
        pair = lax.axis_index("c")
        sid = lax.axis_index("s")
        sl = pl.ds(sid * ROWS_PER_TILE, ROWS_PER_TILE)
        pltpu.sync_copy(
            h_hbm.at[pl.ds(pair * NPAD + sid * ROWS_PER_TILE, ROWS_PER_TILE)],
            acc_sh.at[sl])
        plsc.subcore_barrier()
        for s in range(nb // seg):
            pltpu.sync_copy(src_hbm.at[pair, sid, pl.ds(s * seg, seg)], src_v)
            pltpu.sync_copy(dst_hbm.at[sid, pl.ds(s * seg, seg)], dst_v)
            _pipelined_scatter(h_hbm, src_v, dst_v, rows_v, acc_sh, sem, seg)
        plsc.subcore_barrier()
        pltpu.sync_copy(acc_sh.at[sl], out_hbm.at[pair, sl])

    return body(h_flat, srcb, dst2)


def _tc_l0(agg0, W1, b1, W2, b2, ch_base):
    RB = 1024

    def body(a_ref, w1_ref, b1_ref, w2_ref, b2_ref, out_ref):
        u = a_ref[0] + a_ref[1]
        w1 = w1_ref[...]
        w2 = w2_ref[...]
        for p in range(2):
            halves = []
            for c in (ch_base + 2 * p, ch_base + 2 * p + 1):
                t = u[:, c:c + 1] * w1 + b1_ref[...]
                t = jnp.maximum(t, 0.0)
                halves.append(jnp.dot(t, w2, preferred_element_type=jnp.float32)
                              + b2_ref[...])
            out_ref[p] = jnp.concatenate(halves, axis=1)

    return pl.pallas_call(
        body,
        out_shape=jax.ShapeDtypeStruct((2, NPAD, W), jnp.float32),
        grid=(NPAD // RB,),
        in_specs=[
            pl.BlockSpec((2, RB, W), lambda i: (0, i, 0)),
            pl.BlockSpec((1, H), lambda i: (0, 0)),
            pl.BlockSpec((1, H), lambda i: (0, 0)),
            pl.BlockSpec((H, H), lambda i: (0, 0)),
            pl.BlockSpec((1, H), lambda i: (0, 0)),
        ],
        out_specs=pl.BlockSpec((2, RB, W), lambda i: (0, i, 0)),
    )(agg0, W1, b1, W2, b2)


def _tc_mid(a_flat, W1, b1, W2, b2):
    RB = 2048
    M = 2 * NPAD

    def body(a_ref, w1_ref, b1_ref, w2_ref, b2_ref, out_ref):
        u = a_ref[...]
        halves = []
        for c in range(2):
            uh = u[:, c * H:(c + 1) * H]
            t = jnp.maximum(jnp.dot(uh, w1_ref[...], preferred_element_type=jnp.float32)
                            + b1_ref[...], 0.0)
            halves.append(jnp.dot(t, w2_ref[...], preferred_element_type=jnp.float32)
                          + b2_ref[...])
        out_ref[...] = jnp.concatenate(halves, axis=1)

    return pl.pallas_call(
        body,
        out_shape=jax.ShapeDtypeStruct((M, W), jnp.float32),
        grid=(M // RB,),
        in_specs=[
            pl.BlockSpec((RB, W), lambda i: (i, 0)),
            pl.BlockSpec((H, H), lambda i: (0, 0)),
            pl.BlockSpec((1, H), lambda i: (0, 0)),
            pl.BlockSpec((H, H), lambda i: (0, 0)),
            pl.BlockSpec((1, H), lambda i: (0, 0)),
        ],
        out_specs=pl.BlockSpec((RB, W), lambda i: (i, 0)),
    )(a_flat, W1, b1, W2, b2)


def _tc_rho(h3a, h3b, rW1, rb1, rW2, rb2):
    RB = 1024

    def body(ha_ref, hb_ref, w1_ref, b1_ref, w2_ref, b2_ref, out_ref):
        acc = jnp.zeros((RB, H), jnp.float32)
        for k in range(K):
            h64 = slice((k % 2) * H, (k % 2) * H + H)
            hk = ha_ref[k // 2][:, h64] + hb_ref[k // 2][:, h64]
            acc = acc + jnp.dot(hk, w1_ref[k], preferred_element_type=jnp.float32)
        t = jnp.maximum(acc + b1_ref[...], 0.0)
        out_ref[...] = jnp.dot(t, w2_ref[...], preferred_element_type=jnp.float32) + b2_ref[...]

    return pl.pallas_call(
        body,
        out_shape=jax.ShapeDtypeStruct((NPAD, PE), jnp.float32),
        grid=(NPAD // RB,),
        in_specs=[
            pl.BlockSpec((2, RB, W), lambda i: (0, i, 0)),
            pl.BlockSpec((2, RB, W), lambda i: (0, i, 0)),
            pl.BlockSpec((K, H, H), lambda i: (0, 0, 0)),
            pl.BlockSpec((1, H), lambda i: (0, 0)),
            pl.BlockSpec((H, PE), lambda i: (0, 0)),
            pl.BlockSpec((1, PE), lambda i: (0, 0)),
        ],
        out_specs=pl.BlockSpec((RB, PE), lambda i: (i, 0)),
    )(h3a, h3b, rW1, rb1, rW2, rb2)


def kernel(x, edge_index, batch_index,
           e0_W1, e0_b1, e0_W2, e0_b2,
           e1_W1, e1_b1, e1_W2, e1_b2,
           e2_W1, e2_b1, e2_W2, e2_b2,
           r_W1, r_b1, r_W2, r_b2):
    x2 = x.reshape(N, K)
    xs = jnp.concatenate([x2, -x2], axis=1)
    xpad = jnp.pad(xs, ((0, NPAD - N), (0, W - NCH)))

    src = edge_index[0]
    dst = edge_index[1]
    srcp = jnp.pad(src, (0, EPAD - E))
    dstp = jnp.pad(dst, (0, EPAD - E), constant_values=DUMP)
    src0 = (srcp.reshape(2, -1)
            + (jnp.arange(2, dtype=jnp.int32) * NPAD)[:, None]).reshape(
                2, 16, EPAD // (32 * EB), EB)
    dst0 = dstp.reshape(2, 16, EPAD // (32 * EB), EB)
    pair_off = (jnp.arange(2, dtype=jnp.int32) * NPAD)[:, None]
    srcb = (srcp[None, :] + pair_off).reshape(2, 16, EPAD // (16 * EB), EB)
    dst2 = dstp.reshape(16, EPAD // (16 * EB), EB)
    zeros = jnp.zeros((ROWS_PER_TILE, W), jnp.float32)

    xpad2 = jnp.concatenate([xpad, xpad], axis=0)
    agg0 = _sc_agg0(xpad2, src0, dst0, zeros)

    h3g = []
    for ch_base in (0, 4):
        h1 = _tc_l0(agg0, e0_W1, e0_b1.reshape(1, H), e0_W2,
                    e0_b2.reshape(1, H), ch_base)
        u1 = _sc_agg(h1.reshape(2 * NPAD, W), srcb, dst2)
        h2 = _tc_mid(u1.reshape(2 * NPAD, W),
                     e1_W1, e1_b1.reshape(1, H), e1_W2, e1_b2.reshape(1, H))
        u2 = _sc_agg(h2, srcb, dst2)
        h3 = _tc_mid(u2.reshape(2 * NPAD, W),
                     e2_W1, e2_b1.reshape(1, H), e2_W2, e2_b2.reshape(1, H))
        h3g.append(h3.reshape(2, NPAD, W))

    out = _tc_rho(h3g[0], h3g[1], r_W1.reshape(K, H, H),
                  r_b1.reshape(1, H), r_W2, r_b2.reshape(1, PE))
    return out[:N]


# speedup vs baseline: 1.0567x; 1.0567x over previous
import functools

import jax
import jax.numpy as jnp
from jax import lax
from jax.experimental import pallas as pl
from jax.experimental.pallas import tpu as pltpu
from jax.experimental.pallas import tpu_sc as plsc

N = 10000
E = 160000
K = 4
H = 64
PE = 32

NPAD = 10240
EPAD = 163840
EB = 64
NBUF = 4
NCH = 2 * K
NPAIR = NCH // 2
W = 2 * H
ROWS_PER_TILE = NPAD // 16
DUMP = N


@functools.cache
def _mesh():
    return plsc.VectorSubcoreMesh(core_axis_name="c", subcore_axis_name="s")


def _pipelined_scatter(table_hbm, src_v, dst_v, rows_v, acc_sh, sem, nb):
    for i in range(NBUF):
        pltpu.async_copy(table_hbm.at[src_v.at[i]], rows_v.at[i], sem.at[i])

    @pl.loop(0, nb)
    def _(b):
        buf = lax.rem(b, NBUF)
        pltpu.make_async_copy(table_hbm.at[src_v.at[b]], rows_v.at[buf],
                              sem.at[buf]).wait()
        pltpu.sync_copy(rows_v.at[buf], acc_sh.at[dst_v.at[b]], add=True)

        @pl.when(b + NBUF < nb)
        def _():
            pltpu.async_copy(table_hbm.at[src_v.at[b + NBUF]], rows_v.at[buf],
                             sem.at[buf])


def _sc_agg0(xpad, src0, dst0, zeros):
    nb = src0.shape[2]
    seg = 16

    @functools.partial(
        pl.kernel,
        out_type=jax.ShapeDtypeStruct((2, NPAD, W), jnp.float32),
        mesh=_mesh(),
        scratch_types=[
            pltpu.VMEM((seg, EB), jnp.int32),
            pltpu.VMEM((seg, EB), jnp.int32),
            pltpu.VMEM((NBUF, EB, W), jnp.float32),
            pltpu.VMEM_SHARED((NPAD, W), jnp.float32),
            pltpu.SemaphoreType.DMA((NBUF,)),
        ],
    )
    def body(x_hbm, src_hbm, dst_hbm, z_hbm, out_hbm, src_v, dst_v, rows_v,
             acc_sh, sem):
        cid = lax.axis_index("c")
        sid = lax.axis_index("s")
        sl = pl.ds(sid * ROWS_PER_TILE, ROWS_PER_TILE)

        @pl.when(cid == 0)
        def _():
            pltpu.sync_copy(x_hbm.at[sl], acc_sh.at[sl])

        @pl.when(cid != 0)
        def _():
            pltpu.sync_copy(z_hbm, acc_sh.at[sl])

        plsc.subcore_barrier()
        for s in range(nb // seg):
            pltpu.sync_copy(src_hbm.at[cid, sid, pl.ds(s * seg, seg)], src_v)
            pltpu.sync_copy(dst_hbm.at[cid, sid, pl.ds(s * seg, seg)], dst_v)
            _pipelined_scatter(x_hbm, src_v, dst_v, rows_v, acc_sh, sem, seg)
        plsc.subcore_barrier()
        pltpu.sync_copy(acc_sh.at[sl], out_hbm.at[cid, sl])

    return body(xpad, src0, dst0, zeros)


def _sc_agg(h_flat, srcb, dst2):
    nb = dst2.shape[1]
    seg = 32

    @functools.partial(
        pl.kernel,
        out_type=jax.ShapeDtypeStruct((2, NPAD, W), jnp.float32),
        mesh=_mesh(),
        scratch_types=[
            pltpu.VMEM((seg, EB), jnp.int32),
            pltpu.VMEM((seg, EB), jnp.int32),
            pltpu.VMEM((NBUF, EB, W), jnp.float32),
            pltpu.VMEM_SHARED((NPAD, W), jnp.float32),
            pltpu.SemaphoreType.DMA((NBUF,)),
        ],
    )
    def body(h_hbm, src_hbm, dst_hbm, out_hbm, src_v, dst_v, rows_v,
             acc_sh, sem):
        pair = lax.axis_index("c")
        sid = lax.axis_index("s")
        sl = pl.ds(sid * ROWS_PER_TILE, ROWS_PER_TILE)
        pltpu.sync_copy(
            h_hbm.at[pl.ds(pair * NPAD + sid * ROWS_PER_TILE, ROWS_PER_TILE)],
            acc_sh.at[sl])
        plsc.subcore_barrier()
        for s in range(nb // seg):
            pltpu.sync_copy(src_hbm.at[pair, sid, pl.ds(s * seg, seg)], src_v)
            pltpu.sync_copy(dst_hbm.at[sid, pl.ds(s * seg, seg)], dst_v)
            _pipelined_scatter(h_hbm, src_v, dst_v, rows_v, acc_sh, sem, seg)
        plsc.subcore_barrier()
        pltpu.sync_copy(acc_sh.at[sl], out_hbm.at[pair, sl])

    return body(h_flat, srcb, dst2)


def _tc_l0(agg0, W1, b1, W2, b2, ch_base):
    RB = 1024

    def body(a_ref, w1_ref, b1_ref, w2_ref, b2_ref, out_ref):
        u = a_ref[0] + a_ref[1]
        w1 = w1_ref[...]
        w2 = w2_ref[...]
        for p in range(2):
            halves = []
            for c in (ch_base + 2 * p, ch_base + 2 * p + 1):
                t = u[:, c:c + 1] * w1 + b1_ref[...]
                t = jnp.maximum(t, 0.0)
                halves.append(jnp.dot(t, w2, preferred_element_type=jnp.float32)
                              + b2_ref[...])
            out_ref[p] = jnp.concatenate(halves, axis=1)

    return pl.pallas_call(
        body,
        out_shape=jax.ShapeDtypeStruct((2, NPAD, W), jnp.float32),
        grid=(NPAD // RB,),
        in_specs=[
            pl.BlockSpec((2, RB, W), lambda i: (0, i, 0)),
            pl.BlockSpec((1, H), lambda i: (0, 0)),
            pl.BlockSpec((1, H), lambda i: (0, 0)),
            pl.BlockSpec((H, H), lambda i: (0, 0)),
            pl.BlockSpec((1, H), lambda i: (0, 0)),
        ],
        out_specs=pl.BlockSpec((2, RB, W), lambda i: (0, i, 0)),
    )(agg0, W1, b1, W2, b2)


def _tc_mid(a_flat, W1, b1, W2, b2):
    RB = 2048
    M = 2 * NPAD

    def body(a_ref, w1_ref, b1_ref, w2_ref, b2_ref, out_ref):
        u = a_ref[...]
        halves = []
        for c in range(2):
            uh = u[:, c * H:(c + 1) * H]
            t = jnp.maximum(jnp.dot(uh, w1_ref[...], preferred_element_type=jnp.float32)
                            + b1_ref[...], 0.0)
            halves.append(jnp.dot(t, w2_ref[...], preferred_element_type=jnp.float32)
                          + b2_ref[...])
        out_ref[...] = jnp.concatenate(halves, axis=1)

    return pl.pallas_call(
        body,
        out_shape=jax.ShapeDtypeStruct((M, W), jnp.float32),
        grid=(M // RB,),
        in_specs=[
            pl.BlockSpec((RB, W), lambda i: (i, 0)),
            pl.BlockSpec((H, H), lambda i: (0, 0)),
            pl.BlockSpec((1, H), lambda i: (0, 0)),
            pl.BlockSpec((H, H), lambda i: (0, 0)),
            pl.BlockSpec((1, H), lambda i: (0, 0)),
        ],
        out_specs=pl.BlockSpec((RB, W), lambda i: (i, 0)),
    )(a_flat, W1, b1, W2, b2)


def _tc_rho(h3a, h3b, rW1, rb1, rW2, rb2):
    RB = 1024

    def body(ha_ref, hb_ref, w1_ref, b1_ref, w2_ref, b2_ref, out_ref):
        acc = jnp.zeros((RB, H), jnp.float32)
        for k in range(K):
            h64 = slice((k % 2) * H, (k % 2) * H + H)
            hk = ha_ref[k // 2][:, h64] + hb_ref[k // 2][:, h64]
            acc = acc + jnp.dot(hk, w1_ref[k], preferred_element_type=jnp.float32)
        t = jnp.maximum(acc + b1_ref[...], 0.0)
        out_ref[...] = jnp.dot(t, w2_ref[...], preferred_element_type=jnp.float32) + b2_ref[...]

    return pl.pallas_call(
        body,
        out_shape=jax.ShapeDtypeStruct((NPAD, PE), jnp.float32),
        grid=(NPAD // RB,),
        in_specs=[
            pl.BlockSpec((2, RB, W), lambda i: (0, i, 0)),
            pl.BlockSpec((2, RB, W), lambda i: (0, i, 0)),
            pl.BlockSpec((K, H, H), lambda i: (0, 0, 0)),
            pl.BlockSpec((1, H), lambda i: (0, 0)),
            pl.BlockSpec((H, PE), lambda i: (0, 0)),
            pl.BlockSpec((1, PE), lambda i: (0, 0)),
        ],
        out_specs=pl.BlockSpec((RB, PE), lambda i: (i, 0)),
    )(h3a, h3b, rW1, rb1, rW2, rb2)


def kernel(x, edge_index, batch_index,
           e0_W1, e0_b1, e0_W2, e0_b2,
           e1_W1, e1_b1, e1_W2, e1_b2,
           e2_W1, e2_b1, e2_W2, e2_b2,
           r_W1, r_b1, r_W2, r_b2):
    x2 = x.reshape(N, K)
    xs = jnp.concatenate([x2, -x2], axis=1)
    xpad = jnp.pad(xs, ((0, NPAD - N), (0, W - NCH)))

    src = edge_index[0]
    dst = edge_index[1]
    srcp = jnp.pad(src, (0, EPAD - E))
    dstp = jnp.pad(dst, (0, EPAD - E), constant_values=DUMP)
    src0 = srcp.reshape(2, 16, EPAD // (32 * EB), EB)
    dst0 = dstp.reshape(2, 16, EPAD // (32 * EB), EB)
    pair_off = (jnp.arange(2, dtype=jnp.int32) * NPAD)[:, None]
    srcb = (srcp[None, :] + pair_off).reshape(2, 16, EPAD // (16 * EB), EB)
    dst2 = dstp.reshape(16, EPAD // (16 * EB), EB)
    zeros = jnp.zeros((ROWS_PER_TILE, W), jnp.float32)

    agg0 = _sc_agg0(xpad, src0, dst0, zeros)

    h3g = []
    for ch_base in (0, 4):
        h1 = _tc_l0(agg0, e0_W1, e0_b1.reshape(1, H), e0_W2,
                    e0_b2.reshape(1, H), ch_base)
        u1 = _sc_agg(h1.reshape(2 * NPAD, W), srcb, dst2)
        h2 = _tc_mid(u1.reshape(2 * NPAD, W),
                     e1_W1, e1_b1.reshape(1, H), e1_W2, e1_b2.reshape(1, H))
        u2 = _sc_agg(h2, srcb, dst2)
        h3 = _tc_mid(u2.reshape(2 * NPAD, W),
                     e2_W1, e2_b1.reshape(1, H), e2_W2, e2_b2.reshape(1, H))
        h3g.append(h3.reshape(2, NPAD, W))

    out = _tc_rho(h3g[0], h3g[1], r_W1.reshape(K, H, H),
                  r_b1.reshape(1, H), r_W2, r_b2.reshape(1, PE))
    return out[:N]


# speedup vs baseline: 1.0699x; 1.0125x over previous
import functools

import jax
import jax.numpy as jnp
from jax import lax
from jax.experimental import pallas as pl
from jax.experimental.pallas import tpu as pltpu
from jax.experimental.pallas import tpu_sc as plsc

N = 10000
E = 160000
K = 4
H = 64
PE = 32

NPAD = 10240
EPAD = 163840
EB = 64
NBUF = 4
NCH = 2 * K
NPAIR = NCH // 2
W = 2 * H
ROWS_PER_TILE = NPAD // 16
DUMP = N


@functools.cache
def _mesh():
    return plsc.VectorSubcoreMesh(core_axis_name="c", subcore_axis_name="s")


def _pipelined_scatter(table_hbm, src_v, dst_v, rows_v, acc_sh, sem, nb):
    for i in range(NBUF):
        pltpu.async_copy(table_hbm.at[src_v.at[i]], rows_v.at[i], sem.at[i])

    @pl.loop(0, nb)
    def _(b):
        buf = lax.rem(b, NBUF)
        pltpu.make_async_copy(table_hbm.at[src_v.at[b]], rows_v.at[buf],
                              sem.at[buf]).wait()
        pltpu.sync_copy(rows_v.at[buf], acc_sh.at[dst_v.at[b]], add=True)

        @pl.when(b + NBUF < nb)
        def _():
            pltpu.async_copy(table_hbm.at[src_v.at[b + NBUF]], rows_v.at[buf],
                             sem.at[buf])


def _sc_agg0(xpad, src0, dst0, zeros):
    nb = src0.shape[2]
    seg = 40

    @functools.partial(
        pl.kernel,
        out_type=jax.ShapeDtypeStruct((2, NPAD, W), jnp.float32),
        mesh=_mesh(),
        scratch_types=[
            pltpu.VMEM((seg, EB), jnp.int32),
            pltpu.VMEM((seg, EB), jnp.int32),
            pltpu.VMEM((NBUF, EB, W), jnp.float32),
            pltpu.VMEM_SHARED((NPAD, W), jnp.float32),
            pltpu.SemaphoreType.DMA((NBUF,)),
        ],
    )
    def body(x_hbm, src_hbm, dst_hbm, z_hbm, out_hbm, src_v, dst_v, rows_v,
             acc_sh, sem):
        cid = lax.axis_index("c")
        sid = lax.axis_index("s")
        sl = pl.ds(sid * ROWS_PER_TILE, ROWS_PER_TILE)

        @pl.when(cid == 0)
        def _():
            pltpu.sync_copy(x_hbm.at[sl], acc_sh.at[sl])

        @pl.when(cid != 0)
        def _():
            pltpu.sync_copy(z_hbm, acc_sh.at[sl])

        plsc.subcore_barrier()
        for s in range(nb // seg):
            pltpu.sync_copy(src_hbm.at[cid, sid, pl.ds(s * seg, seg)], src_v)
            pltpu.sync_copy(dst_hbm.at[cid, sid, pl.ds(s * seg, seg)], dst_v)
            _pipelined_scatter(x_hbm, src_v, dst_v, rows_v, acc_sh, sem, seg)
        plsc.subcore_barrier()
        pltpu.sync_copy(acc_sh.at[sl], out_hbm.at[cid, sl])

    return body(xpad, src0, dst0, zeros)


def _sc_agg(h_flat, srcb, dst2):
    nb = dst2.shape[1]
    seg = 40

    @functools.partial(
        pl.kernel,
        out_type=jax.ShapeDtypeStruct((2, NPAD, W), jnp.float32),
        mesh=_mesh(),
        scratch_types=[
            pltpu.VMEM((seg, EB), jnp.int32),
            pltpu.VMEM((seg, EB), jnp.int32),
            pltpu.VMEM((NBUF, EB, W), jnp.float32),
            pltpu.VMEM_SHARED((NPAD, W), jnp.float32),
            pltpu.SemaphoreType.DMA((NBUF,)),
        ],
    )
    def body(h_hbm, src_hbm, dst_hbm, out_hbm, src_v, dst_v, rows_v,
             acc_sh, sem):
        pair = lax.axis_index("c")
        sid = lax.axis_index("s")
        sl = pl.ds(sid * ROWS_PER_TILE, ROWS_PER_TILE)
        pltpu.sync_copy(
            h_hbm.at[pl.ds(pair * NPAD + sid * ROWS_PER_TILE, ROWS_PER_TILE)],
            acc_sh.at[sl])
        plsc.subcore_barrier()
        for s in range(nb // seg):
            pltpu.sync_copy(src_hbm.at[pair, sid, pl.ds(s * seg, seg)], src_v)
            pltpu.sync_copy(dst_hbm.at[sid, pl.ds(s * seg, seg)], dst_v)
            _pipelined_scatter(h_hbm, src_v, dst_v, rows_v, acc_sh, sem, seg)
        plsc.subcore_barrier()
        pltpu.sync_copy(acc_sh.at[sl], out_hbm.at[pair, sl])

    return body(h_flat, srcb, dst2)


def _tc_l0(agg0, W1, b1, W2, b2, ch_base):
    RB = 1024

    def body(a_ref, w1_ref, b1_ref, w2_ref, b2_ref, out_ref):
        u = a_ref[0] + a_ref[1]
        w1 = w1_ref[...]
        w2 = w2_ref[...]
        for p in range(2):
            halves = []
            for c in (ch_base + 2 * p, ch_base + 2 * p + 1):
                t = u[:, c:c + 1] * w1 + b1_ref[...]
                t = jnp.maximum(t, 0.0)
                halves.append(jnp.dot(t, w2, preferred_element_type=jnp.float32)
                              + b2_ref[...])
            out_ref[p] = jnp.concatenate(halves, axis=1)

    return pl.pallas_call(
        body,
        out_shape=jax.ShapeDtypeStruct((2, NPAD, W), jnp.float32),
        grid=(NPAD // RB,),
        in_specs=[
            pl.BlockSpec((2, RB, W), lambda i: (0, i, 0)),
            pl.BlockSpec((1, H), lambda i: (0, 0)),
            pl.BlockSpec((1, H), lambda i: (0, 0)),
            pl.BlockSpec((H, H), lambda i: (0, 0)),
            pl.BlockSpec((1, H), lambda i: (0, 0)),
        ],
        out_specs=pl.BlockSpec((2, RB, W), lambda i: (0, i, 0)),
    )(agg0, W1, b1, W2, b2)


def _tc_mid(a_flat, W1, b1, W2, b2):
    RB = 2048
    M = 2 * NPAD

    def body(a_ref, w1_ref, b1_ref, w2_ref, b2_ref, out_ref):
        u = a_ref[...]
        halves = []
        for c in range(2):
            uh = u[:, c * H:(c + 1) * H]
            t = jnp.maximum(jnp.dot(uh, w1_ref[...], preferred_element_type=jnp.float32)
                            + b1_ref[...], 0.0)
            halves.append(jnp.dot(t, w2_ref[...], preferred_element_type=jnp.float32)
                          + b2_ref[...])
        out_ref[...] = jnp.concatenate(halves, axis=1)

    return pl.pallas_call(
        body,
        out_shape=jax.ShapeDtypeStruct((M, W), jnp.float32),
        grid=(M // RB,),
        in_specs=[
            pl.BlockSpec((RB, W), lambda i: (i, 0)),
            pl.BlockSpec((H, H), lambda i: (0, 0)),
            pl.BlockSpec((1, H), lambda i: (0, 0)),
            pl.BlockSpec((H, H), lambda i: (0, 0)),
            pl.BlockSpec((1, H), lambda i: (0, 0)),
        ],
        out_specs=pl.BlockSpec((RB, W), lambda i: (i, 0)),
    )(a_flat, W1, b1, W2, b2)


def _tc_rho(h3a, h3b, rW1, rb1, rW2, rb2):
    RB = 1024

    def body(ha_ref, hb_ref, w1_ref, b1_ref, w2_ref, b2_ref, out_ref):
        acc = jnp.zeros((RB, H), jnp.float32)
        for k in range(K):
            h64 = slice((k % 2) * H, (k % 2) * H + H)
            hk = ha_ref[k // 2][:, h64] + hb_ref[k // 2][:, h64]
            acc = acc + jnp.dot(hk, w1_ref[k], preferred_element_type=jnp.float32)
        t = jnp.maximum(acc + b1_ref[...], 0.0)
        out_ref[...] = jnp.dot(t, w2_ref[...], preferred_element_type=jnp.float32) + b2_ref[...]

    return pl.pallas_call(
        body,
        out_shape=jax.ShapeDtypeStruct((NPAD, PE), jnp.float32),
        grid=(NPAD // RB,),
        in_specs=[
            pl.BlockSpec((2, RB, W), lambda i: (0, i, 0)),
            pl.BlockSpec((2, RB, W), lambda i: (0, i, 0)),
            pl.BlockSpec((K, H, H), lambda i: (0, 0, 0)),
            pl.BlockSpec((1, H), lambda i: (0, 0)),
            pl.BlockSpec((H, PE), lambda i: (0, 0)),
            pl.BlockSpec((1, PE), lambda i: (0, 0)),
        ],
        out_specs=pl.BlockSpec((RB, PE), lambda i: (i, 0)),
    )(h3a, h3b, rW1, rb1, rW2, rb2)


def kernel(x, edge_index, batch_index,
           e0_W1, e0_b1, e0_W2, e0_b2,
           e1_W1, e1_b1, e1_W2, e1_b2,
           e2_W1, e2_b1, e2_W2, e2_b2,
           r_W1, r_b1, r_W2, r_b2):
    x2 = x.reshape(N, K)
    xs = jnp.concatenate([x2, -x2], axis=1)
    xpad = jnp.pad(xs, ((0, NPAD - N), (0, W - NCH)))

    src = edge_index[0]
    dst = edge_index[1]
    srcp = jnp.pad(src, (0, EPAD - E))
    dstp = jnp.pad(dst, (0, EPAD - E), constant_values=DUMP)
    src0 = srcp.reshape(2, 16, EPAD // (32 * EB), EB)
    dst0 = dstp.reshape(2, 16, EPAD // (32 * EB), EB)
    pair_off = (jnp.arange(2, dtype=jnp.int32) * NPAD)[:, None]
    srcb = (srcp[None, :] + pair_off).reshape(2, 16, EPAD // (16 * EB), EB)
    dst2 = dstp.reshape(16, EPAD // (16 * EB), EB)
    zeros = jnp.zeros((ROWS_PER_TILE, W), jnp.float32)

    agg0 = _sc_agg0(xpad, src0, dst0, zeros)

    h3g = []
    for ch_base in (0, 4):
        h1 = _tc_l0(agg0, e0_W1, e0_b1.reshape(1, H), e0_W2,
                    e0_b2.reshape(1, H), ch_base)
        u1 = _sc_agg(h1.reshape(2 * NPAD, W), srcb, dst2)
        h2 = _tc_mid(u1.reshape(2 * NPAD, W),
                     e1_W1, e1_b1.reshape(1, H), e1_W2, e1_b2.reshape(1, H))
        u2 = _sc_agg(h2, srcb, dst2)
        h3 = _tc_mid(u2.reshape(2 * NPAD, W),
                     e2_W1, e2_b1.reshape(1, H), e2_W2, e2_b2.reshape(1, H))
        h3g.append(h3.reshape(2, NPAD, W))

    out = _tc_rho(h3g[0], h3g[1], r_W1.reshape(K, H, H),
                  r_b1.reshape(1, H), r_W2, r_b2.reshape(1, PE))
    return out[:N]
